# baseline (speedup 1.0000x reference)
import jax
import jax.numpy as jnp
from jax import lax
from jax.experimental import pallas as pl
from jax.experimental.pallas import tpu as pltpu
from jax.experimental.pallas import tpu_sc as plsc

N_V = 1000
N_D = 64
BATCH = 4096
HIST = 200
B_TOTAL = BATCH * HIST

NC = 2
NS = 16
NW = NC * NS

CHUNK = 128
K = 4
GROUP = CHUNK * K

SC_GROUPS = 33
B_SC = NW * SC_GROUPS * GROUP
CHUNKS_PER_W = SC_GROUPS * K
N_SC_CHUNKS = B_SC // CHUNK

RBLK = 1024
VPAD = 1024
B_TC = B_TOTAL - B_SC
TC_BLOCKS = B_TC // RBLK


def _sc_body(idx_hbm, table_hbm, out_hbm, idx_v, rows_v, table_sh,
             gsems, wsems):
  wid = lax.axis_index("s") * NC + lax.axis_index("c")
  chunk_base = wid * CHUNKS_PER_W
  row_base = chunk_base * CHUNK

  @pl.when(lax.axis_index("s") == 0)
  def _():
    pltpu.sync_copy(table_hbm, table_sh)

  pltpu.sync_copy(idx_hbm.at[pl.ds(chunk_base, CHUNKS_PER_W)], idx_v)
  plsc.subcore_barrier()

  def fire_group(g, pg):
    for b in range(K):
      pltpu.async_copy(
          table_sh.at[idx_v.at[g * K + b]],
          rows_v.at[pg].at[pl.ds(b * CHUNK, CHUNK)],
          gsems.at[pg])

  def drain_group(g, pg):
    for b in range(K):
      pltpu.make_async_copy(
          table_sh.at[idx_v.at[g * K + b]],
          rows_v.at[pg].at[pl.ds(b * CHUNK, CHUNK)],
          gsems.at[pg]).wait()

  def start_write(g, pg):
    pltpu.async_copy(
        rows_v.at[pg], out_hbm.at[pl.ds(row_base + g * GROUP, GROUP)],
        wsems.at[pg])

  def wait_write(g, pg):
    pltpu.make_async_copy(
        rows_v.at[pg], out_hbm.at[pl.ds(row_base + g * GROUP, GROUP)],
        wsems.at[pg]).wait()

  fire_group(0, 0)

  def body(g, _):
    pg = lax.rem(g, 2)
    og = 1 - pg

    @pl.when(g + 1 < SC_GROUPS)
    def _():
      @pl.when(g >= 1)
      def _():
        wait_write(g - 1, og)
      fire_group(g + 1, og)

    drain_group(g, pg)
    start_write(g, pg)
    return 0

  lax.fori_loop(0, SC_GROUPS, body, 0)

  wait_write(SC_GROUPS - 2, lax.rem(SC_GROUPS - 2, 2))
  wait_write(SC_GROUPS - 1, lax.rem(SC_GROUPS - 1, 2))


def _tc_body(idx_ref, w_ref, out_ref):
  idxv = idx_ref[0]
  cols = lax.broadcasted_iota(jnp.int32, (RBLK, VPAD), 1)
  oh = (idxv == cols).astype(jnp.bfloat16)
  out_ref[...] = jnp.dot(oh, w_ref[...],
                         preferred_element_type=jnp.float32)


@jax.jit
def kernel(input_, W):
  idx_flat = input_.reshape(B_TOTAL)

  sc_run = pl.kernel(
      _sc_body,
      out_type=jax.ShapeDtypeStruct((B_SC, N_D), jnp.float32),
      mesh=plsc.VectorSubcoreMesh(core_axis_name="c", subcore_axis_name="s"),
      scratch_types=[
          pltpu.VMEM((CHUNKS_PER_W, CHUNK), jnp.int32),
          pltpu.VMEM((2, GROUP, N_D), jnp.float32),
          pltpu.VMEM_SHARED((N_V, N_D), jnp.float32),
          pltpu.SemaphoreType.DMA((2,)),
          pltpu.SemaphoreType.DMA((2,)),
      ],
      compiler_params=pltpu.CompilerParams(use_tc_tiling_on_sc=False),
  )
  sc_out = sc_run(idx_flat[:B_SC].reshape(N_SC_CHUNKS, CHUNK), W)

  tc_out = pl.pallas_call(
      _tc_body,
      grid=(TC_BLOCKS,),
      in_specs=[
          pl.BlockSpec((1, RBLK, 1), lambda i: (i, 0, 0)),
          pl.BlockSpec((VPAD, N_D), lambda i: (0, 0)),
      ],
      out_specs=pl.BlockSpec((RBLK, N_D), lambda i: (i, 0)),
      out_shape=jax.ShapeDtypeStruct((B_TC, N_D), jnp.float32),
  )(idx_flat[B_SC:].reshape(TC_BLOCKS, RBLK, 1),
    jnp.zeros((VPAD, N_D), jnp.bfloat16).at[:N_V].set(
        W.astype(jnp.bfloat16)))

  out = jnp.concatenate([sc_out, tc_out], axis=0)
  return out.reshape(BATCH, HIST, N_D)


# speedup vs baseline: 2.6156x; 2.6156x over previous
import jax
import jax.numpy as jnp
from jax import lax
from jax.experimental import pallas as pl
from jax.experimental.pallas import tpu as pltpu
from jax.experimental.pallas import tpu_sc as plsc

N_V = 1000
N_D = 64
BATCH = 4096
HIST = 200

NC = 2
NS = 16
NW = NC * NS
L = 16

B_TOTAL = BATCH * HIST
ROWS_PER_W = B_TOTAL // NW
GROUP = 256
N_GROUPS = ROWS_PER_W // GROUP
BLOCKS = GROUP // L
GROUP_WORDS = GROUP * N_D
CHUNKS_PER_GROUP = GROUP // 128

T_STRIDE = 8 * 32 * 8 * 128
CH_STRIDE = 32 * 8 * 128
BH_STRIDE = 8 * 128
SG_CH = CHUNKS_PER_GROUP * 8 * 128
SG_BH = 8 * 128


def _embed_body(idx_hbm, table_hbm, out_hbm, idx_v, table_v, rows_v, wsems):
  wid = lax.axis_index("s") * NC + lax.axis_index("c")
  p_base = wid * (ROWS_PER_W // 128)

  pltpu.sync_copy(table_hbm, table_v)
  pltpu.sync_copy(idx_hbm.at[pl.ds(wid * ROWS_PER_W, ROWS_PER_W)], idx_v)

  lanes = lax.iota(jnp.int32, L)
  diag = [(lanes + d) & (L - 1) for d in range(L)]
  sgoff = [((diag[d] >> 3) * SG_CH) + ((diag[d] & 7) * 128) + lanes
           for d in range(L)]

  def hbm_off(g, ch):
    p0 = p_base + g * CHUNKS_PER_GROUP
    t = p0 >> 5
    bh0 = p0 & 31
    return t * T_STRIDE + ch * CH_STRIDE + bh0 * BH_STRIDE

  def write_group(g, pg):
    for ch in range(8):
      pltpu.async_copy(
          rows_v.at[pl.ds(pg * GROUP_WORDS + ch * SG_CH, SG_CH)],
          out_hbm.at[pl.ds(hbm_off(g, ch), SG_CH)],
          wsems.at[pg])

  def wait_group(g, pg):
    for ch in range(8):
      pltpu.make_async_copy(
          rows_v.at[pl.ds(pg * GROUP_WORDS + ch * SG_CH, SG_CH)],
          out_hbm.at[pl.ds(hbm_off(g, ch), SG_CH)],
          wsems.at[pg]).wait()

  def expand_group(g, pg):
    pg_words = pg * GROUP_WORDS

    @plsc.parallel_loop(0, BLOCKS, unroll=2)
    def _(i):
      v = idx_v[pl.ds(g * GROUP + i * L, L)]
      src_base = v * N_D
      dst_base = pg_words + (i >> 3) * SG_BH + (i & 7) * L
      for bc in range(N_D // L):
        for d in range(L):
          col = plsc.load_gather(table_v, [src_base + (diag[d] + bc * L)])
          plsc.store_scatter(
              rows_v, [sgoff[d] + (dst_base + bc * 2 * SG_CH)], col)

  @pl.loop(0, N_GROUPS)
  def _(g):
    pg = lax.rem(g, 2)

    @pl.when(g >= 2)
    def _():
      wait_group(g - 2, pg)

    expand_group(g, pg)
    write_group(g, pg)

  for g in (N_GROUPS - 2, N_GROUPS - 1):
    wait_group(g, g % 2)


@jax.jit
def kernel(input_, W):
  idx_t = input_.T.reshape(B_TOTAL)
  table_flat = W.reshape(N_V * N_D)
  run = pl.kernel(
      _embed_body,
      out_type=jax.ShapeDtypeStruct((B_TOTAL * N_D,), jnp.float32),
      mesh=plsc.VectorSubcoreMesh(core_axis_name="c", subcore_axis_name="s"),
      scratch_types=[
          pltpu.VMEM((ROWS_PER_W,), jnp.int32),
          pltpu.VMEM((N_V * N_D,), jnp.float32),
          pltpu.VMEM((2 * GROUP_WORDS,), jnp.float32),
          pltpu.SemaphoreType.DMA((2,)),
      ],
      compiler_params=pltpu.CompilerParams(
          use_tc_tiling_on_sc=False, needs_layout_passes=False,
          disable_bounds_checks=True),
  )
  out = run(idx_t, table_flat)
  a = out.reshape(HIST, 8, 32, 8, 128)
  return a.transpose(2, 4, 0, 1, 3).reshape(BATCH, HIST, N_D)


# speedup vs baseline: 8.6881x; 3.3217x over previous
import jax
import jax.numpy as jnp
from jax import lax
from jax.experimental import pallas as pl
from jax.experimental.pallas import tpu as pltpu
from jax.experimental.pallas import tpu_sc as plsc

N_V = 1000
N_D = 64
BATCH = 4096
HIST = 200

NC = 2
NS = 16
NW = NC * NS
L = 16

B_TOTAL = BATCH * HIST
ROWS_PER_W = B_TOTAL // NW
GROUP = 256
N_GROUPS = ROWS_PER_W // GROUP
BLOCKS = GROUP // L
GROUP_WORDS = GROUP * N_D
CHUNKS_PER_GROUP = GROUP // 128

T_STRIDE = 8 * 32 * 8 * 128
CH_STRIDE = 32 * 8 * 128
BH_STRIDE = 8 * 128
SG_CH = CHUNKS_PER_GROUP * 8 * 128
SG_BH = 8 * 128


def _embed_body(idx_hbm, table_hbm, out_hbm, idx_v, table_v, rows_v, wsems):
  wid = lax.axis_index("s") * NC + lax.axis_index("c")
  p_base = wid * (ROWS_PER_W // 128)

  pltpu.sync_copy(table_hbm, table_v)
  pltpu.sync_copy(idx_hbm.at[pl.ds(wid * ROWS_PER_W, ROWS_PER_W)], idx_v)

  lanes = lax.iota(jnp.int32, L)
  diag = [(lanes + d) & (L - 1) for d in range(L)]
  sgoff = [((diag[d] >> 3) * SG_CH) + ((diag[d] & 7) * 128) + lanes
           for d in range(L)]

  def hbm_off(g, ch):
    p0 = p_base + g * CHUNKS_PER_GROUP
    t = p0 >> 5
    bh0 = p0 & 31
    return t * T_STRIDE + ch * CH_STRIDE + bh0 * BH_STRIDE

  def write_group(g, pg):
    for ch in range(8):
      pltpu.async_copy(
          rows_v.at[pl.ds(pg * GROUP_WORDS + ch * SG_CH, SG_CH)],
          out_hbm.at[pl.ds(hbm_off(g, ch), SG_CH)],
          wsems.at[pg])

  def wait_group(g, pg):
    for ch in range(8):
      pltpu.make_async_copy(
          rows_v.at[pl.ds(pg * GROUP_WORDS + ch * SG_CH, SG_CH)],
          out_hbm.at[pl.ds(hbm_off(g, ch), SG_CH)],
          wsems.at[pg]).wait()

  def expand_group(g, pg):
    pg_words = pg * GROUP_WORDS

    @plsc.parallel_loop(0, BLOCKS, unroll=4)
    def _(i):
      v = idx_v[pl.ds(g * GROUP + i * L, L)]
      src_base = v * N_D
      dst_base = pg_words + (i >> 3) * SG_BH + (i & 7) * L
      for bc in range(N_D // L):
        for d in range(L):
          col = plsc.load_gather(table_v, [src_base + (diag[d] + bc * L)])
          plsc.store_scatter(
              rows_v, [sgoff[d] + (dst_base + bc * 2 * SG_CH)], col)

  @pl.loop(0, N_GROUPS)
  def _(g):
    pg = lax.rem(g, 2)

    @pl.when(g >= 2)
    def _():
      wait_group(g - 2, pg)

    expand_group(g, pg)
    write_group(g, pg)

  for g in (N_GROUPS - 2, N_GROUPS - 1):
    wait_group(g, g % 2)


@jax.jit
def kernel(input_, W):
  idx_t = input_.T.reshape(B_TOTAL)
  table_flat = W.reshape(N_V * N_D)
  run = pl.kernel(
      _embed_body,
      out_type=jax.ShapeDtypeStruct((B_TOTAL * N_D,), jnp.float32),
      mesh=plsc.VectorSubcoreMesh(core_axis_name="c", subcore_axis_name="s"),
      scratch_types=[
          pltpu.VMEM((ROWS_PER_W,), jnp.int32),
          pltpu.VMEM((N_V * N_D,), jnp.float32),
          pltpu.VMEM((2 * GROUP_WORDS,), jnp.float32),
          pltpu.SemaphoreType.DMA((2,)),
      ],
      compiler_params=pltpu.CompilerParams(
          use_tc_tiling_on_sc=False, needs_layout_passes=False,
          disable_bounds_checks=True),
  )
  out = run(idx_t, table_flat)
  a = out.reshape(HIST, 8, 32, 8, 128)
  return a.transpose(2, 4, 0, 1, 3).reshape(BATCH, HIST, N_D)
